# Initial kernel scaffold; baseline (speedup 1.0000x reference)
import functools

import jax
import jax.numpy as jnp
from jax import lax
from jax.experimental import pallas as pl
from jax.experimental.pallas import tpu as pltpu
from jax.experimental.pallas import tpu_sc as plsc

N = 10000
E = 320000
D_IN = 128
D_HID = 64
D_OUT = 16
K = 10
ALPHA = 0.1

NCORES = 2
NSUB = 16
NTILES = NCORES * NSUB
NPAD = 10016
RPS = NPAD // NSUB
TRASH = 10008
CHUNK = 128
CPT = 80
EPT = CHUNK * CPT
EPAD = EPT * NTILES

_mesh = plsc.VectorSubcoreMesh(core_axis_name="c", subcore_axis_name="s")


def _zero_rows(zbuf):
    @pl.loop(0, zbuf.shape[0])
    def _(i):
        zbuf[i, :] = jnp.zeros((16,), jnp.float32)


@functools.partial(
    pl.kernel,
    out_type=jax.ShapeDtypeStruct((NCORES, NPAD, D_OUT), jnp.float32),
    mesh=_mesh,
    scratch_types=[
        pltpu.VMEM((CPT, CHUNK), jnp.int32),
        pltpu.VMEM((CPT, CHUNK), jnp.int32),
        pltpu.VMEM((CHUNK, D_OUT), jnp.float32),
        pltpu.VMEM((RPS, D_OUT), jnp.float32),
        pltpu.VMEM_SHARED((NPAD, D_OUT), jnp.float32),
    ],
)
def _edge_scatter(src_hbm, dst_hbm, s_hbm, out_hbm, src_v, dst_v, gbuf, zbuf, t_sh):
    c = lax.axis_index("c")
    sub = lax.axis_index("s")
    w = c * NSUB + sub
    pltpu.sync_copy(src_hbm.at[w], src_v)
    pltpu.sync_copy(dst_hbm.at[w], dst_v)
    _zero_rows(zbuf)
    pltpu.sync_copy(zbuf, t_sh.at[pl.ds(sub * RPS, RPS)])
    plsc.subcore_barrier()

    @pl.loop(0, CPT)
    def _(j):
        pltpu.sync_copy(s_hbm.at[src_v.at[j]], gbuf)
        pltpu.sync_copy(gbuf, t_sh.at[dst_v.at[j]], add=True)

    plsc.subcore_barrier()
    pltpu.sync_copy(t_sh.at[pl.ds(sub * RPS, RPS)], zbuf)
    pltpu.sync_copy(zbuf, out_hbm.at[c].at[pl.ds(sub * RPS, RPS)])


@functools.partial(
    pl.kernel,
    out_type=jax.ShapeDtypeStruct((NCORES, NPAD, D_OUT), jnp.float32),
    mesh=_mesh,
    scratch_types=[
        pltpu.VMEM((CPT, CHUNK), jnp.int32),
        pltpu.VMEM((CHUNK, D_OUT), jnp.float32),
        pltpu.VMEM((RPS, D_OUT), jnp.float32),
        pltpu.VMEM_SHARED((NPAD, D_OUT), jnp.float32),
    ],
)
def _deg_scatter(dst_hbm, out_hbm, dst_v, obuf, zbuf, t_sh):
    c = lax.axis_index("c")
    sub = lax.axis_index("s")
    w = c * NSUB + sub
    pltpu.sync_copy(dst_hbm.at[w], dst_v)

    @pl.loop(0, CHUNK)
    def _(i):
        obuf[i, :] = jnp.ones((16,), jnp.float32)

    _zero_rows(zbuf)
    pltpu.sync_copy(zbuf, t_sh.at[pl.ds(sub * RPS, RPS)])
    plsc.subcore_barrier()

    @pl.loop(0, CPT)
    def _(j):
        pltpu.sync_copy(obuf, t_sh.at[dst_v.at[j]], add=True)

    plsc.subcore_barrier()
    pltpu.sync_copy(t_sh.at[pl.ds(sub * RPS, RPS)], zbuf)
    pltpu.sync_copy(zbuf, out_hbm.at[c].at[pl.ds(sub * RPS, RPS)])


def _mlp_body(x_ref, w1_ref, b1_ref, w2_ref, b2_ref, o_ref):
    h = jnp.dot(x_ref[...], w1_ref[...], preferred_element_type=jnp.float32)
    h = jnp.maximum(h + b1_ref[...], 0.0)
    o_ref[...] = (
        jnp.dot(h, w2_ref[...], preferred_element_type=jnp.float32) + b2_ref[...]
    )


def _mlp(xp, W1, b1, W2, b2):
    return pl.pallas_call(
        _mlp_body,
        out_shape=jax.ShapeDtypeStruct((NPAD, D_OUT), jnp.float32),
    )(xp, W1, b1.reshape(1, D_HID), W2, b2.reshape(1, D_OUT))


def _prep_body(pd_ref, h0_ref, s0_ref, cs_ref, bs_ref, ch_ref, bh_ref):
    pd = pd_ref[...]
    d = pd[0, :, 0:1] + pd[1, :, 0:1] + 1.0
    dinv = lax.rsqrt(d)
    rows = lax.broadcasted_iota(jnp.int32, (NPAD, 1), 0)
    dinv = jnp.where(rows < N, dinv, 0.0)
    h0 = h0_ref[...]
    s0_ref[...] = dinv * h0
    cs_ref[...] = jnp.broadcast_to((1.0 - ALPHA) * dinv * dinv, (NPAD, D_OUT))
    bs_ref[...] = (ALPHA * dinv) * h0
    ch_ref[...] = jnp.broadcast_to((1.0 - ALPHA) * dinv, (NPAD, D_OUT))
    bh_ref[...] = ALPHA * h0


def _prep(pdeg, h0):
    sh = jax.ShapeDtypeStruct((NPAD, D_OUT), jnp.float32)
    return pl.pallas_call(
        _prep_body,
        out_shape=(sh, sh, sh, sh, sh),
    )(pdeg, h0)


def _update_body(p_ref, s_ref, c_ref, b_ref, o_ref):
    p = p_ref[...]
    o_ref[...] = c_ref[...] * (p[0] + p[1] + s_ref[...]) + b_ref[...]


def _update(p, s, coeff, bias):
    return pl.pallas_call(
        _update_body,
        out_shape=jax.ShapeDtypeStruct((NPAD, D_OUT), jnp.float32),
    )(p, s, coeff, bias)


def kernel(x, edge_index, W1, b1, W2, b2):
    xp = jnp.pad(x, ((0, NPAD - N), (0, 0)))
    pad = jnp.full((EPAD - E,), TRASH, jnp.int32)
    src3 = jnp.concatenate([edge_index[0], pad]).reshape(NTILES, CPT, CHUNK)
    dst3 = jnp.concatenate([edge_index[1], pad]).reshape(NTILES, CPT, CHUNK)

    h0 = _mlp(xp, W1, b1, W2, b2)
    pdeg = _deg_scatter(dst3)
    s0, cs, bs, ch, bh = _prep(pdeg, h0)

    s = s0
    for k in range(K):
        p = _edge_scatter(src3, dst3, s)
        if k < K - 1:
            s = _update(p, s, cs, bs)
        else:
            out = _update(p, s, ch, bh)
    return out[:N]


# speedup vs baseline: 18.7118x; 18.7118x over previous
import functools

import jax
import jax.numpy as jnp
from jax import lax
from jax.experimental import pallas as pl
from jax.experimental.pallas import tpu as pltpu
from jax.experimental.pallas import tpu_sc as plsc

N = 10000
E = 320000
D_IN = 128
D_HID = 64
D_OUT = 16
K = 10
ALPHA = 0.1

NCORES = 2
NSUB = 16
NTILES = NCORES * NSUB
NPAD = 10112
RPS = NPAD // NSUB
TRASH = 10008
CHUNK = 128
CPT = 80
EPT = CHUNK * CPT
EPAD = EPT * NTILES

_mesh = plsc.VectorSubcoreMesh(core_axis_name="c", subcore_axis_name="s")
_sc_params = pltpu.CompilerParams(use_tc_tiling_on_sc=False)


def _zero_rows(zbuf):
    @pl.loop(0, zbuf.shape[0])
    def _(i):
        zbuf[i, :] = jnp.zeros((16,), jnp.float32)


@functools.partial(
    pl.kernel,
    out_type=jax.ShapeDtypeStruct((NCORES, NPAD, D_OUT), jnp.float32),
    mesh=_mesh,
    scratch_types=[
        pltpu.VMEM((CPT, CHUNK), jnp.int32),
        pltpu.VMEM((CPT, CHUNK), jnp.int32),
        pltpu.VMEM((CHUNK, D_OUT), jnp.float32),
        pltpu.VMEM((RPS, D_OUT), jnp.float32),
        pltpu.VMEM_SHARED((NPAD, D_OUT), jnp.float32),
    ],
    compiler_params=_sc_params,
)
def _edge_scatter(src_hbm, dst_hbm, s_hbm, out_hbm, src_v, dst_v, gbuf, zbuf, t_sh):
    c = lax.axis_index("c")
    sub = lax.axis_index("s")
    w = c * NSUB + sub
    pltpu.sync_copy(src_hbm.at[w], src_v)
    pltpu.sync_copy(dst_hbm.at[w], dst_v)
    _zero_rows(zbuf)
    pltpu.sync_copy(zbuf, t_sh.at[pl.ds(pl.multiple_of(sub * RPS, 8), RPS)])
    plsc.subcore_barrier()

    @pl.loop(0, CPT)
    def _(j):
        pltpu.sync_copy(s_hbm.at[src_v.at[j]], gbuf)
        pltpu.sync_copy(gbuf, t_sh.at[dst_v.at[j]], add=True)

    plsc.subcore_barrier()
    pltpu.sync_copy(t_sh.at[pl.ds(pl.multiple_of(sub * RPS, 8), RPS)], zbuf)
    pltpu.sync_copy(zbuf, out_hbm.at[c].at[pl.ds(pl.multiple_of(sub * RPS, 8), RPS)])


@functools.partial(
    pl.kernel,
    out_type=jax.ShapeDtypeStruct((NCORES, NPAD, D_OUT), jnp.float32),
    mesh=_mesh,
    scratch_types=[
        pltpu.VMEM((CPT, CHUNK), jnp.int32),
        pltpu.VMEM((CHUNK, D_OUT), jnp.float32),
        pltpu.VMEM((RPS, D_OUT), jnp.float32),
        pltpu.VMEM_SHARED((NPAD, D_OUT), jnp.float32),
    ],
    compiler_params=_sc_params,
)
def _deg_scatter(dst_hbm, out_hbm, dst_v, obuf, zbuf, t_sh):
    c = lax.axis_index("c")
    sub = lax.axis_index("s")
    w = c * NSUB + sub
    pltpu.sync_copy(dst_hbm.at[w], dst_v)

    @pl.loop(0, CHUNK)
    def _(i):
        obuf[i, :] = jnp.ones((16,), jnp.float32)

    _zero_rows(zbuf)
    pltpu.sync_copy(zbuf, t_sh.at[pl.ds(pl.multiple_of(sub * RPS, 8), RPS)])
    plsc.subcore_barrier()

    @pl.loop(0, CPT)
    def _(j):
        pltpu.sync_copy(obuf, t_sh.at[dst_v.at[j]], add=True)

    plsc.subcore_barrier()
    pltpu.sync_copy(t_sh.at[pl.ds(pl.multiple_of(sub * RPS, 8), RPS)], zbuf)
    pltpu.sync_copy(zbuf, out_hbm.at[c].at[pl.ds(pl.multiple_of(sub * RPS, 8), RPS)])


def _mlp_body(x_ref, w1_ref, b1_ref, w2_ref, b2_ref, o_ref):
    h = jnp.dot(x_ref[...], w1_ref[...], preferred_element_type=jnp.float32)
    h = jnp.maximum(h + b1_ref[...], 0.0)
    o_ref[...] = (
        jnp.dot(h, w2_ref[...], preferred_element_type=jnp.float32) + b2_ref[...]
    )


def _mlp(xp, W1, b1, W2, b2):
    return pl.pallas_call(
        _mlp_body,
        out_shape=jax.ShapeDtypeStruct((NPAD, D_OUT), jnp.float32),
    )(xp, W1, b1.reshape(1, D_HID), W2, b2.reshape(1, D_OUT))


def _prep_body(pd_ref, h0_ref, s0_ref, cs_ref, bs_ref, ch_ref, bh_ref):
    pd = pd_ref[...]
    d = pd[0, :, 0:1] + pd[1, :, 0:1] + 1.0
    dinv = lax.rsqrt(d)
    rows = lax.broadcasted_iota(jnp.int32, (NPAD, 1), 0)
    dinv = jnp.where(rows < N, dinv, 0.0)
    h0 = h0_ref[...]
    s0_ref[...] = dinv * h0
    cs_ref[...] = jnp.broadcast_to((1.0 - ALPHA) * dinv * dinv, (NPAD, D_OUT))
    bs_ref[...] = (ALPHA * dinv) * h0
    ch_ref[...] = jnp.broadcast_to((1.0 - ALPHA) * dinv, (NPAD, D_OUT))
    bh_ref[...] = ALPHA * h0


def _prep(pdeg, h0):
    sh = jax.ShapeDtypeStruct((NPAD, D_OUT), jnp.float32)
    return pl.pallas_call(
        _prep_body,
        out_shape=(sh, sh, sh, sh, sh),
    )(pdeg, h0)


def _update_body(p_ref, s_ref, c_ref, b_ref, o_ref):
    p = p_ref[...]
    o_ref[...] = c_ref[...] * (p[0] + p[1] + s_ref[...]) + b_ref[...]


def _update(p, s, coeff, bias):
    return pl.pallas_call(
        _update_body,
        out_shape=jax.ShapeDtypeStruct((NPAD, D_OUT), jnp.float32),
    )(p, s, coeff, bias)


def kernel(x, edge_index, W1, b1, W2, b2):
    xp = jnp.pad(x, ((0, NPAD - N), (0, 0)))
    pad = jnp.full((EPAD - E,), TRASH, jnp.int32)
    src3 = jnp.concatenate([edge_index[0], pad]).reshape(NTILES, CPT, CHUNK)
    dst3 = jnp.concatenate([edge_index[1], pad]).reshape(NTILES, CPT, CHUNK)

    h0 = _mlp(xp, W1, b1, W2, b2)
    pdeg = _deg_scatter(dst3)
    s0, cs, bs, ch, bh = _prep(pdeg, h0)

    s = s0
    for k in range(K):
        p = _edge_scatter(src3, dst3, s)
        if k < K - 1:
            s = _update(p, s, cs, bs)
        else:
            out = _update(p, s, ch, bh)
    return out[:N]


# speedup vs baseline: 25.6875x; 1.3728x over previous
import functools

import jax
import jax.numpy as jnp
from jax import lax
from jax.experimental import pallas as pl
from jax.experimental.pallas import tpu as pltpu
from jax.experimental.pallas import tpu_sc as plsc

N = 10000
E = 320000
D_IN = 128
D_HID = 64
D_OUT = 16
K = 10
ALPHA = 0.1

NCORES = 2
NSUB = 16
NTILES = NCORES * NSUB
NPAD = 10112
RPS = NPAD // NSUB
TRASH = 10008
CHUNK = 128
CPT = 80
EPT = CHUNK * CPT
EPAD = EPT * NTILES

_mesh = plsc.VectorSubcoreMesh(core_axis_name="c", subcore_axis_name="s")
_sc_params = pltpu.CompilerParams(use_tc_tiling_on_sc=False)


def _zero_rows(zbuf):
    @pl.loop(0, zbuf.shape[0])
    def _(i):
        zbuf[i, :] = jnp.zeros((16,), jnp.float32)


@functools.partial(
    pl.kernel,
    out_type=jax.ShapeDtypeStruct((NCORES, NPAD, D_OUT), jnp.float32),
    mesh=_mesh,
    scratch_types=[
        pltpu.VMEM((CPT, CHUNK), jnp.int32),
        pltpu.VMEM((CPT, CHUNK), jnp.int32),
        pltpu.VMEM((CHUNK, D_OUT), jnp.float32),
        pltpu.VMEM((CHUNK, D_OUT), jnp.float32),
        pltpu.VMEM((RPS, D_OUT), jnp.float32),
        pltpu.VMEM_SHARED((NPAD, D_OUT), jnp.float32),
        pltpu.SemaphoreType.DMA,
        pltpu.SemaphoreType.DMA,
    ],
    compiler_params=_sc_params,
)
def _edge_scatter(src_hbm, dst_hbm, s_hbm, out_hbm, src_v, dst_v, gA, gB, zbuf,
                  t_sh, semA, semB):
    c = lax.axis_index("c")
    sub = lax.axis_index("s")
    w = c * NSUB + sub
    pltpu.sync_copy(src_hbm.at[w], src_v)
    pltpu.sync_copy(dst_hbm.at[w], dst_v)
    _zero_rows(zbuf)
    pltpu.sync_copy(zbuf, t_sh.at[pl.ds(pl.multiple_of(sub * RPS, 8), RPS)])
    plsc.subcore_barrier()

    pltpu.async_copy(s_hbm.at[src_v.at[0]], gA, semA)
    pltpu.async_copy(s_hbm.at[src_v.at[1]], gB, semB)

    @pl.loop(0, CPT // 2 - 1)
    def _(i):
        j = i * 2
        pltpu.make_async_copy(s_hbm.at[src_v.at[j]], gA, semA).wait()
        pltpu.sync_copy(gA, t_sh.at[dst_v.at[j]], add=True)
        pltpu.async_copy(s_hbm.at[src_v.at[j + 2]], gA, semA)
        pltpu.make_async_copy(s_hbm.at[src_v.at[j + 1]], gB, semB).wait()
        pltpu.sync_copy(gB, t_sh.at[dst_v.at[j + 1]], add=True)
        pltpu.async_copy(s_hbm.at[src_v.at[j + 3]], gB, semB)

    jt = CPT - 2
    pltpu.make_async_copy(s_hbm.at[src_v.at[jt]], gA, semA).wait()
    pltpu.sync_copy(gA, t_sh.at[dst_v.at[jt]], add=True)
    pltpu.make_async_copy(s_hbm.at[src_v.at[jt + 1]], gB, semB).wait()
    pltpu.sync_copy(gB, t_sh.at[dst_v.at[jt + 1]], add=True)

    plsc.subcore_barrier()
    pltpu.sync_copy(t_sh.at[pl.ds(pl.multiple_of(sub * RPS, 8), RPS)], zbuf)
    pltpu.sync_copy(zbuf, out_hbm.at[c].at[pl.ds(pl.multiple_of(sub * RPS, 8), RPS)])


@functools.partial(
    pl.kernel,
    out_type=jax.ShapeDtypeStruct((NCORES, NPAD, D_OUT), jnp.float32),
    mesh=_mesh,
    scratch_types=[
        pltpu.VMEM((CPT, CHUNK), jnp.int32),
        pltpu.VMEM((CHUNK, D_OUT), jnp.float32),
        pltpu.VMEM((RPS, D_OUT), jnp.float32),
        pltpu.VMEM_SHARED((NPAD, D_OUT), jnp.float32),
    ],
    compiler_params=_sc_params,
)
def _deg_scatter(dst_hbm, out_hbm, dst_v, obuf, zbuf, t_sh):
    c = lax.axis_index("c")
    sub = lax.axis_index("s")
    w = c * NSUB + sub
    pltpu.sync_copy(dst_hbm.at[w], dst_v)

    @pl.loop(0, CHUNK)
    def _(i):
        obuf[i, :] = jnp.ones((16,), jnp.float32)

    _zero_rows(zbuf)
    pltpu.sync_copy(zbuf, t_sh.at[pl.ds(pl.multiple_of(sub * RPS, 8), RPS)])
    plsc.subcore_barrier()

    @pl.loop(0, CPT)
    def _(j):
        pltpu.sync_copy(obuf, t_sh.at[dst_v.at[j]], add=True)

    plsc.subcore_barrier()
    pltpu.sync_copy(t_sh.at[pl.ds(pl.multiple_of(sub * RPS, 8), RPS)], zbuf)
    pltpu.sync_copy(zbuf, out_hbm.at[c].at[pl.ds(pl.multiple_of(sub * RPS, 8), RPS)])


def _mlp_body(x_ref, w1_ref, b1_ref, w2_ref, b2_ref, o_ref):
    h = jnp.dot(x_ref[...], w1_ref[...], preferred_element_type=jnp.float32)
    h = jnp.maximum(h + b1_ref[...], 0.0)
    o_ref[...] = (
        jnp.dot(h, w2_ref[...], preferred_element_type=jnp.float32) + b2_ref[...]
    )


def _mlp(xp, W1, b1, W2, b2):
    return pl.pallas_call(
        _mlp_body,
        out_shape=jax.ShapeDtypeStruct((NPAD, D_OUT), jnp.float32),
    )(xp, W1, b1.reshape(1, D_HID), W2, b2.reshape(1, D_OUT))


def _prep_body(pd_ref, h0_ref, s0_ref, cs_ref, bs_ref, ch_ref, bh_ref):
    pd = pd_ref[...]
    d = pd[0, :, 0:1] + pd[1, :, 0:1] + 1.0
    dinv = lax.rsqrt(d)
    rows = lax.broadcasted_iota(jnp.int32, (NPAD, 1), 0)
    dinv = jnp.where(rows < N, dinv, 0.0)
    h0 = h0_ref[...]
    s0_ref[...] = dinv * h0
    cs_ref[...] = jnp.broadcast_to((1.0 - ALPHA) * dinv * dinv, (NPAD, D_OUT))
    bs_ref[...] = (ALPHA * dinv) * h0
    ch_ref[...] = jnp.broadcast_to((1.0 - ALPHA) * dinv, (NPAD, D_OUT))
    bh_ref[...] = ALPHA * h0


def _prep(pdeg, h0):
    sh = jax.ShapeDtypeStruct((NPAD, D_OUT), jnp.float32)
    return pl.pallas_call(
        _prep_body,
        out_shape=(sh, sh, sh, sh, sh),
    )(pdeg, h0)


def _update_body(p_ref, s_ref, c_ref, b_ref, o_ref):
    p = p_ref[...]
    o_ref[...] = c_ref[...] * (p[0] + p[1] + s_ref[...]) + b_ref[...]


def _update(p, s, coeff, bias):
    return pl.pallas_call(
        _update_body,
        out_shape=jax.ShapeDtypeStruct((NPAD, D_OUT), jnp.float32),
    )(p, s, coeff, bias)


def kernel(x, edge_index, W1, b1, W2, b2):
    xp = jnp.pad(x, ((0, NPAD - N), (0, 0)))
    pad = jnp.full((EPAD - E,), TRASH, jnp.int32)
    src3 = jnp.concatenate([edge_index[0], pad]).reshape(NTILES, CPT, CHUNK)
    dst3 = jnp.concatenate([edge_index[1], pad]).reshape(NTILES, CPT, CHUNK)

    h0 = _mlp(xp, W1, b1, W2, b2)
    pdeg = _deg_scatter(dst3)
    s0, cs, bs, ch, bh = _prep(pdeg, h0)

    s = s0
    for k in range(K):
        p = _edge_scatter(src3, dst3, s)
        if k < K - 1:
            s = _update(p, s, cs, bs)
        else:
            out = _update(p, s, ch, bh)
    return out[:N]


# speedup vs baseline: 53.1035x; 2.0673x over previous
import functools

import jax
import jax.numpy as jnp
from jax import lax
from jax.experimental import pallas as pl
from jax.experimental.pallas import tpu as pltpu
from jax.experimental.pallas import tpu_sc as plsc

N = 10000
E = 320000
D_IN = 128
D_HID = 64
D_OUT = 16
K = 10
ALPHA = 0.1

NCORES = 2
NSUB = 16
NTILES = NCORES * NSUB
NPAD = 10112
RPS = NPAD // NSUB
TRASH = 10008
CHUNK = 128
CPT = 80
EPT = CHUNK * CPT
EPAD = EPT * NTILES

_mesh = plsc.VectorSubcoreMesh(core_axis_name="c", subcore_axis_name="s")
_sc_params = pltpu.CompilerParams(use_tc_tiling_on_sc=False)


def _zero_rows(zbuf):
    @pl.loop(0, zbuf.shape[0])
    def _(i):
        zbuf[i, :] = jnp.zeros((16,), jnp.float32)


@functools.partial(
    pl.kernel,
    out_type=(
        jax.ShapeDtypeStruct((NCORES, NPAD, D_OUT), jnp.float32),
        jax.ShapeDtypeStruct((NPAD, D_OUT), jnp.float32),
    ),
    mesh=_mesh,
    scratch_types=[
        pltpu.VMEM((CPT, CHUNK), jnp.int32),
        pltpu.VMEM((CPT, CHUNK), jnp.int32),
        pltpu.VMEM((CHUNK, D_OUT), jnp.float32),
        pltpu.VMEM((CHUNK, D_OUT), jnp.float32),
        pltpu.VMEM((RPS, D_OUT), jnp.float32),
        pltpu.VMEM((RPS, D_OUT), jnp.float32),
        pltpu.VMEM((RPS, D_OUT), jnp.float32),
        pltpu.VMEM((RPS, D_OUT), jnp.float32),
        pltpu.VMEM((RPS, D_OUT), jnp.float32),
        pltpu.VMEM((RPS, D_OUT), jnp.float32),
        pltpu.VMEM_SHARED((NPAD, D_OUT), jnp.float32),
        pltpu.VMEM_SHARED((NPAD, D_OUT), jnp.float32),
        pltpu.SemaphoreType.DMA,
        pltpu.SemaphoreType.DMA,
    ],
    compiler_params=_sc_params,
)
def _hop(src_hbm, dst_hbm, p_hbm, sp_hbm, cs_hbm, bs_hbm, pout_hbm, sout_hbm,
         src_v, dst_v, gA, gB, sbuf, a0, a1, a2, a3, a4, s_sh, t_sh,
         semA, semB):
    c = lax.axis_index("c")
    sub = lax.axis_index("s")
    w = c * NSUB + sub
    rng = pl.ds(pl.multiple_of(sub * RPS, 8), RPS)
    pltpu.sync_copy(src_hbm.at[w], src_v)
    pltpu.sync_copy(dst_hbm.at[w], dst_v)
    pltpu.sync_copy(p_hbm.at[0].at[rng], a0)
    pltpu.sync_copy(p_hbm.at[1].at[rng], a1)
    pltpu.sync_copy(sp_hbm.at[rng], a2)
    pltpu.sync_copy(cs_hbm.at[rng], a3)
    pltpu.sync_copy(bs_hbm.at[rng], a4)

    @pl.loop(0, RPS)
    def _(i):
        sbuf[i, :] = a3[i, :] * (a0[i, :] + a1[i, :] + a2[i, :]) + a4[i, :]

    pltpu.sync_copy(sbuf, s_sh.at[rng])
    _zero_rows(a0)
    pltpu.sync_copy(a0, t_sh.at[rng])

    @pl.when(c == 0)
    def _():
        pltpu.sync_copy(sbuf, sout_hbm.at[rng])

    plsc.subcore_barrier()

    pltpu.async_copy(s_sh.at[src_v.at[0]], gA, semA)
    pltpu.async_copy(s_sh.at[src_v.at[1]], gB, semB)

    @pl.loop(0, CPT // 2 - 1)
    def _(i):
        j = i * 2
        pltpu.make_async_copy(s_sh.at[src_v.at[j]], gA, semA).wait()
        pltpu.sync_copy(gA, t_sh.at[dst_v.at[j]], add=True)
        pltpu.async_copy(s_sh.at[src_v.at[j + 2]], gA, semA)
        pltpu.make_async_copy(s_sh.at[src_v.at[j + 1]], gB, semB).wait()
        pltpu.sync_copy(gB, t_sh.at[dst_v.at[j + 1]], add=True)
        pltpu.async_copy(s_sh.at[src_v.at[j + 3]], gB, semB)

    jt = CPT - 2
    pltpu.make_async_copy(s_sh.at[src_v.at[jt]], gA, semA).wait()
    pltpu.sync_copy(gA, t_sh.at[dst_v.at[jt]], add=True)
    pltpu.make_async_copy(s_sh.at[src_v.at[jt + 1]], gB, semB).wait()
    pltpu.sync_copy(gB, t_sh.at[dst_v.at[jt + 1]], add=True)

    plsc.subcore_barrier()
    pltpu.sync_copy(t_sh.at[rng], sbuf)
    pltpu.sync_copy(sbuf, pout_hbm.at[c].at[rng])


@functools.partial(
    pl.kernel,
    out_type=jax.ShapeDtypeStruct((NCORES, NPAD, D_OUT), jnp.float32),
    mesh=_mesh,
    scratch_types=[
        pltpu.VMEM((CPT, CHUNK), jnp.int32),
        pltpu.VMEM((CHUNK, D_OUT), jnp.float32),
        pltpu.VMEM((RPS, D_OUT), jnp.float32),
        pltpu.VMEM_SHARED((NPAD, D_OUT), jnp.float32),
    ],
    compiler_params=_sc_params,
)
def _deg_scatter(dst_hbm, out_hbm, dst_v, obuf, zbuf, t_sh):
    c = lax.axis_index("c")
    sub = lax.axis_index("s")
    w = c * NSUB + sub
    pltpu.sync_copy(dst_hbm.at[w], dst_v)

    @pl.loop(0, CHUNK)
    def _(i):
        obuf[i, :] = jnp.ones((16,), jnp.float32)

    _zero_rows(zbuf)
    pltpu.sync_copy(zbuf, t_sh.at[pl.ds(pl.multiple_of(sub * RPS, 8), RPS)])
    plsc.subcore_barrier()

    @pl.loop(0, CPT)
    def _(j):
        pltpu.sync_copy(obuf, t_sh.at[dst_v.at[j]], add=True)

    plsc.subcore_barrier()
    pltpu.sync_copy(t_sh.at[pl.ds(pl.multiple_of(sub * RPS, 8), RPS)], zbuf)
    pltpu.sync_copy(zbuf, out_hbm.at[c].at[pl.ds(pl.multiple_of(sub * RPS, 8), RPS)])


def _mlp_body(x_ref, w1_ref, b1_ref, w2_ref, b2_ref, o_ref):
    h = jnp.dot(x_ref[...], w1_ref[...], preferred_element_type=jnp.float32)
    h = jnp.maximum(h + b1_ref[...], 0.0)
    o_ref[...] = (
        jnp.dot(h, w2_ref[...], preferred_element_type=jnp.float32) + b2_ref[...]
    )


def _mlp(xp, W1, b1, W2, b2):
    return pl.pallas_call(
        _mlp_body,
        out_shape=jax.ShapeDtypeStruct((NPAD, D_OUT), jnp.float32),
    )(xp, W1, b1.reshape(1, D_HID), W2, b2.reshape(1, D_OUT))


def _prep_body(pd_ref, h0_ref, s0_ref, cs_ref, bs_ref, ch_ref, bh_ref):
    pd = pd_ref[...]
    d = pd[0, :, 0:1] + pd[1, :, 0:1] + 1.0
    dinv = lax.rsqrt(d)
    rows = lax.broadcasted_iota(jnp.int32, (NPAD, 1), 0)
    dinv = jnp.where(rows < N, dinv, 0.0)
    h0 = h0_ref[...]
    s0_ref[...] = dinv * h0
    cs_ref[...] = jnp.broadcast_to((1.0 - ALPHA) * dinv * dinv, (NPAD, D_OUT))
    bs_ref[...] = (ALPHA * dinv) * h0
    ch_ref[...] = jnp.broadcast_to((1.0 - ALPHA) * dinv, (NPAD, D_OUT))
    bh_ref[...] = ALPHA * h0


def _prep(pdeg, h0):
    sh = jax.ShapeDtypeStruct((NPAD, D_OUT), jnp.float32)
    return pl.pallas_call(
        _prep_body,
        out_shape=(sh, sh, sh, sh, sh),
    )(pdeg, h0)


def _update_body(p_ref, s_ref, c_ref, b_ref, o_ref):
    p = p_ref[...]
    o_ref[...] = c_ref[...] * (p[0] + p[1] + s_ref[...]) + b_ref[...]


def _update(p, s, coeff, bias):
    return pl.pallas_call(
        _update_body,
        out_shape=jax.ShapeDtypeStruct((NPAD, D_OUT), jnp.float32),
    )(p, s, coeff, bias)


def kernel(x, edge_index, W1, b1, W2, b2):
    xp = jnp.pad(x, ((0, NPAD - N), (0, 0)))
    pad = jnp.full((EPAD - E,), TRASH, jnp.int32)
    src3 = jnp.concatenate([edge_index[0], pad]).reshape(NTILES, CPT, CHUNK)
    dst3 = jnp.concatenate([edge_index[1], pad]).reshape(NTILES, CPT, CHUNK)

    h0 = _mlp(xp, W1, b1, W2, b2)
    pdeg = _deg_scatter(dst3)
    s0, cs, bs, ch, bh = _prep(pdeg, h0)

    zp = jnp.zeros((NCORES, NPAD, D_OUT), jnp.float32)
    zs = jnp.zeros((NPAD, D_OUT), jnp.float32)
    p, s = _hop(src3, dst3, zp, zs, cs, s0)
    for _ in range(K - 1):
        p, s = _hop(src3, dst3, p, s, cs, bs)
    return _update(p, s, ch, bh)[:N]


# speedup vs baseline: 57.5312x; 1.0834x over previous
import functools

import jax
import jax.numpy as jnp
from jax import lax
from jax.experimental import pallas as pl
from jax.experimental.pallas import tpu as pltpu
from jax.experimental.pallas import tpu_sc as plsc

N = 10000
E = 320000
D_IN = 128
D_HID = 64
D_OUT = 16
K = 10
ALPHA = 0.1

NCORES = 2
NSUB = 16
NTILES = NCORES * NSUB
NPAD = 10112
RPS = NPAD // NSUB
TRASH = 10008
CHUNK = 128
CPT = 80
EPT = CHUNK * CPT
EPAD = EPT * NTILES

_mesh = plsc.VectorSubcoreMesh(core_axis_name="c", subcore_axis_name="s")
_sc_params = pltpu.CompilerParams(use_tc_tiling_on_sc=False)


def _zero_rows(zbuf):
    @pl.loop(0, zbuf.shape[0], step=4)
    def _(i):
        for u in range(4):
            zbuf[i + u, :] = jnp.zeros((16,), jnp.float32)


@functools.partial(
    pl.kernel,
    out_type=(
        jax.ShapeDtypeStruct((NCORES, NPAD, D_OUT), jnp.float32),
        jax.ShapeDtypeStruct((NPAD, D_OUT), jnp.float32),
    ),
    mesh=_mesh,
    scratch_types=[
        pltpu.VMEM((CPT, CHUNK), jnp.int32),
        pltpu.VMEM((CPT, CHUNK), jnp.int32),
        [pltpu.VMEM((CHUNK, D_OUT), jnp.float32) for _ in range(4)],
        pltpu.VMEM((RPS, D_OUT), jnp.float32),
        pltpu.VMEM((RPS, D_OUT), jnp.float32),
        pltpu.VMEM((RPS, D_OUT), jnp.float32),
        pltpu.VMEM((RPS, D_OUT), jnp.float32),
        pltpu.VMEM((RPS, D_OUT), jnp.float32),
        pltpu.VMEM((RPS, D_OUT), jnp.float32),
        pltpu.VMEM((RPS, D_OUT), jnp.float32),
        pltpu.VMEM_SHARED((NPAD, D_OUT), jnp.float32),
        pltpu.VMEM_SHARED((NPAD, D_OUT), jnp.float32),
        [pltpu.SemaphoreType.DMA for _ in range(4)],
        [pltpu.SemaphoreType.DMA for _ in range(4)],
        pltpu.SemaphoreType.DMA,
    ],
    compiler_params=_sc_params,
)
def _hop(src_hbm, dst_hbm, p_hbm, sp_hbm, cs_hbm, bs_hbm, pout_hbm, sout_hbm,
         src_v, dst_v, g, sbuf, a0, a1, a2, a3, a4, zq, s_sh, t_sh,
         semG, semS, semP):
    c = lax.axis_index("c")
    sub = lax.axis_index("s")
    w = c * NSUB + sub
    rng = pl.ds(pl.multiple_of(sub * RPS, 8), RPS)
    idx_stage = [(src_hbm.at[w], src_v), (dst_hbm.at[w], dst_v)]
    row_stage = [
        (p_hbm.at[0].at[rng], a0), (p_hbm.at[1].at[rng], a1),
        (sp_hbm.at[rng], a2), (cs_hbm.at[rng], a3), (bs_hbm.at[rng], a4),
    ]
    for s_ref, d_ref in idx_stage:
        pltpu.async_copy(s_ref, d_ref, semG[0])
    for s_ref, d_ref in row_stage:
        pltpu.async_copy(s_ref, d_ref, semP)
    _zero_rows(zq)
    pltpu.sync_copy(zq, t_sh.at[rng])
    for s_ref, d_ref in row_stage:
        pltpu.make_async_copy(s_ref, d_ref, semP).wait()

    @pl.loop(0, RPS, step=4)
    def _(i):
        for u in range(4):
            sbuf[i + u, :] = (
                a3[i + u, :] * (a0[i + u, :] + a1[i + u, :] + a2[i + u, :])
                + a4[i + u, :]
            )

    for s_ref, d_ref in idx_stage:
        pltpu.make_async_copy(s_ref, d_ref, semG[0]).wait()

    pltpu.sync_copy(sbuf, s_sh.at[rng])

    @pl.when(c == 0)
    def _():
        pltpu.sync_copy(sbuf, sout_hbm.at[rng])

    plsc.subcore_barrier()

    for b in range(4):
        pltpu.async_copy(s_sh.at[src_v.at[b]], g[b], semG[b])
    for b in range(4):
        pltpu.make_async_copy(s_sh.at[src_v.at[b]], g[b], semG[b]).wait()
        pltpu.async_copy(g[b], t_sh.at[dst_v.at[b]], semS[b], add=True)

    @pl.loop(1, CPT // 4)
    def _(i):
        j = i * 4
        for b in range(4):
            pltpu.make_async_copy(g[b], t_sh.at[dst_v.at[j + b - 4]],
                                  semS[b]).wait()
            pltpu.async_copy(s_sh.at[src_v.at[j + b]], g[b], semG[b])
        for b in range(4):
            pltpu.make_async_copy(s_sh.at[src_v.at[j + b]], g[b],
                                  semG[b]).wait()
            pltpu.async_copy(g[b], t_sh.at[dst_v.at[j + b]], semS[b],
                             add=True)

    for b in range(4):
        pltpu.make_async_copy(g[b], t_sh.at[dst_v.at[CPT + b - 4]],
                              semS[b]).wait()

    plsc.subcore_barrier()
    pltpu.sync_copy(t_sh.at[rng], sbuf)
    pltpu.sync_copy(sbuf, pout_hbm.at[c].at[rng])


@functools.partial(
    pl.kernel,
    out_type=jax.ShapeDtypeStruct((NCORES, NPAD, D_OUT), jnp.float32),
    mesh=_mesh,
    scratch_types=[
        pltpu.VMEM((CPT, CHUNK), jnp.int32),
        pltpu.VMEM((CHUNK, D_OUT), jnp.float32),
        pltpu.VMEM((RPS, D_OUT), jnp.float32),
        pltpu.VMEM_SHARED((NPAD, D_OUT), jnp.float32),
        pltpu.SemaphoreType.DMA,
    ],
    compiler_params=_sc_params,
)
def _deg_scatter(dst_hbm, out_hbm, dst_v, obuf, zbuf, t_sh, sem):
    c = lax.axis_index("c")
    sub = lax.axis_index("s")
    w = c * NSUB + sub
    pltpu.sync_copy(dst_hbm.at[w], dst_v)

    @pl.loop(0, CHUNK)
    def _(i):
        obuf[i, :] = jnp.ones((16,), jnp.float32)

    _zero_rows(zbuf)
    pltpu.sync_copy(zbuf, t_sh.at[pl.ds(pl.multiple_of(sub * RPS, 8), RPS)])
    plsc.subcore_barrier()

    @pl.loop(0, CPT // 8)
    def _(i):
        j = i * 8
        for b in range(8):
            pltpu.async_copy(obuf, t_sh.at[dst_v.at[j + b]], sem, add=True)
        for b in range(8):
            pltpu.make_async_copy(obuf, t_sh.at[dst_v.at[j + b]],
                                  sem).wait()

    plsc.subcore_barrier()
    pltpu.sync_copy(t_sh.at[pl.ds(pl.multiple_of(sub * RPS, 8), RPS)], zbuf)
    pltpu.sync_copy(zbuf, out_hbm.at[c].at[pl.ds(pl.multiple_of(sub * RPS, 8), RPS)])


def _mlp_body(x_ref, w1_ref, b1_ref, w2_ref, b2_ref, o_ref):
    h = jnp.dot(x_ref[...], w1_ref[...], preferred_element_type=jnp.float32)
    h = jnp.maximum(h + b1_ref[...], 0.0)
    o_ref[...] = (
        jnp.dot(h, w2_ref[...], preferred_element_type=jnp.float32) + b2_ref[...]
    )


def _mlp(xp, W1, b1, W2, b2):
    return pl.pallas_call(
        _mlp_body,
        out_shape=jax.ShapeDtypeStruct((NPAD, D_OUT), jnp.float32),
    )(xp, W1, b1.reshape(1, D_HID), W2, b2.reshape(1, D_OUT))


def _prep_body(pd_ref, h0_ref, s0_ref, cs_ref, bs_ref, ch_ref, bh_ref):
    pd = pd_ref[...]
    d = pd[0, :, 0:1] + pd[1, :, 0:1] + 1.0
    dinv = lax.rsqrt(d)
    rows = lax.broadcasted_iota(jnp.int32, (NPAD, 1), 0)
    dinv = jnp.where(rows < N, dinv, 0.0)
    h0 = h0_ref[...]
    s0_ref[...] = dinv * h0
    cs_ref[...] = jnp.broadcast_to((1.0 - ALPHA) * dinv * dinv, (NPAD, D_OUT))
    bs_ref[...] = (ALPHA * dinv) * h0
    ch_ref[...] = jnp.broadcast_to((1.0 - ALPHA) * dinv, (NPAD, D_OUT))
    bh_ref[...] = ALPHA * h0


def _prep(pdeg, h0):
    sh = jax.ShapeDtypeStruct((NPAD, D_OUT), jnp.float32)
    return pl.pallas_call(
        _prep_body,
        out_shape=(sh, sh, sh, sh, sh),
    )(pdeg, h0)


def _update_body(p_ref, s_ref, c_ref, b_ref, o_ref):
    p = p_ref[...]
    o_ref[...] = c_ref[...] * (p[0] + p[1] + s_ref[...]) + b_ref[...]


def _update(p, s, coeff, bias):
    return pl.pallas_call(
        _update_body,
        out_shape=jax.ShapeDtypeStruct((NPAD, D_OUT), jnp.float32),
    )(p, s, coeff, bias)


def kernel(x, edge_index, W1, b1, W2, b2):
    xp = jnp.pad(x, ((0, NPAD - N), (0, 0)))
    pad = jnp.full((EPAD - E,), TRASH, jnp.int32)
    src3 = jnp.concatenate([edge_index[0], pad]).reshape(NTILES, CPT, CHUNK)
    dst3 = jnp.concatenate([edge_index[1], pad]).reshape(NTILES, CPT, CHUNK)

    h0 = _mlp(xp, W1, b1, W2, b2)
    pdeg = _deg_scatter(dst3)
    s0, cs, bs, ch, bh = _prep(pdeg, h0)

    zp = jnp.zeros((NCORES, NPAD, D_OUT), jnp.float32)
    zs = jnp.zeros((NPAD, D_OUT), jnp.float32)
    p, s = _hop(src3, dst3, zp, zs, cs, s0)
    for _ in range(K - 1):
        p, s = _hop(src3, dst3, p, s, cs, bs)
    return _update(p, s, ch, bh)[:N]


# speedup vs baseline: 58.2544x; 1.0126x over previous
import functools

import jax
import jax.numpy as jnp
from jax import lax
from jax.experimental import pallas as pl
from jax.experimental.pallas import tpu as pltpu
from jax.experimental.pallas import tpu_sc as plsc

N = 10000
E = 320000
D_IN = 128
D_HID = 64
D_OUT = 16
K = 10
ALPHA = 0.1

NCORES = 2
NSUB = 16
NTILES = NCORES * NSUB
NPAD = 10112
RPS = NPAD // NSUB
TRASH = 10008
CHUNK = 128
CPT = 80
EPT = CHUNK * CPT
EPAD = EPT * NTILES

_mesh = plsc.VectorSubcoreMesh(core_axis_name="c", subcore_axis_name="s")
_sc_params = pltpu.CompilerParams(use_tc_tiling_on_sc=False)


def _zero_rows(zbuf):
    @pl.loop(0, zbuf.shape[0], step=4)
    def _(i):
        for u in range(4):
            zbuf[i + u, :] = jnp.zeros((16,), jnp.float32)


@functools.partial(
    pl.kernel,
    out_type=(
        jax.ShapeDtypeStruct((NCORES, NPAD, D_OUT), jnp.float32),
        jax.ShapeDtypeStruct((NPAD, D_OUT), jnp.float32),
    ),
    mesh=_mesh,
    scratch_types=[
        pltpu.VMEM((CPT, CHUNK), jnp.int32),
        pltpu.VMEM((CPT, CHUNK), jnp.int32),
        [pltpu.VMEM((CHUNK, D_OUT), jnp.float32) for _ in range(8)],
        pltpu.VMEM((RPS, D_OUT), jnp.float32),
        pltpu.VMEM((RPS, D_OUT), jnp.float32),
        pltpu.VMEM((RPS, D_OUT), jnp.float32),
        pltpu.VMEM((RPS, D_OUT), jnp.float32),
        pltpu.VMEM((RPS, D_OUT), jnp.float32),
        pltpu.VMEM((RPS, D_OUT), jnp.float32),
        pltpu.VMEM((RPS, D_OUT), jnp.float32),
        pltpu.VMEM_SHARED((NPAD, D_OUT), jnp.float32),
        pltpu.VMEM_SHARED((NPAD, D_OUT), jnp.float32),
        [pltpu.SemaphoreType.DMA for _ in range(8)],
        [pltpu.SemaphoreType.DMA for _ in range(8)],
        pltpu.SemaphoreType.DMA,
    ],
    compiler_params=_sc_params,
)
def _hop(src_hbm, dst_hbm, p_hbm, sp_hbm, cs_hbm, bs_hbm, pout_hbm, sout_hbm,
         src_v, dst_v, g, sbuf, a0, a1, a2, a3, a4, zq, s_sh, t_sh,
         semG, semS, semP):
    c = lax.axis_index("c")
    sub = lax.axis_index("s")
    w = c * NSUB + sub
    rng = pl.ds(pl.multiple_of(sub * RPS, 8), RPS)
    idx_stage = [(src_hbm.at[w], src_v), (dst_hbm.at[w], dst_v)]
    row_stage = [
        (p_hbm.at[0].at[rng], a0), (p_hbm.at[1].at[rng], a1),
        (sp_hbm.at[rng], a2), (cs_hbm.at[rng], a3), (bs_hbm.at[rng], a4),
    ]
    for s_ref, d_ref in idx_stage:
        pltpu.async_copy(s_ref, d_ref, semG[0])
    for s_ref, d_ref in row_stage:
        pltpu.async_copy(s_ref, d_ref, semP)
    _zero_rows(zq)
    pltpu.sync_copy(zq, t_sh.at[rng])
    for s_ref, d_ref in row_stage:
        pltpu.make_async_copy(s_ref, d_ref, semP).wait()

    @pl.loop(0, RPS, step=4)
    def _(i):
        for u in range(4):
            sbuf[i + u, :] = (
                a3[i + u, :] * (a0[i + u, :] + a1[i + u, :] + a2[i + u, :])
                + a4[i + u, :]
            )

    for s_ref, d_ref in idx_stage:
        pltpu.make_async_copy(s_ref, d_ref, semG[0]).wait()

    pltpu.sync_copy(sbuf, s_sh.at[rng])

    @pl.when(c == 0)
    def _():
        pltpu.sync_copy(sbuf, sout_hbm.at[rng])

    plsc.subcore_barrier()

    NB = 8
    for b in range(NB):
        pltpu.async_copy(s_sh.at[src_v.at[b]], g[b], semG[b])
    for b in range(NB):
        pltpu.make_async_copy(s_sh.at[src_v.at[b]], g[b], semG[b]).wait()
        pltpu.async_copy(g[b], t_sh.at[dst_v.at[b]], semS[b], add=True)

    @pl.loop(1, CPT // NB)
    def _(i):
        j = i * NB
        for b in range(NB):
            pltpu.make_async_copy(g[b], t_sh.at[dst_v.at[j + b - NB]],
                                  semS[b]).wait()
            pltpu.async_copy(s_sh.at[src_v.at[j + b]], g[b], semG[b])
        for b in range(NB):
            pltpu.make_async_copy(s_sh.at[src_v.at[j + b]], g[b],
                                  semG[b]).wait()
            pltpu.async_copy(g[b], t_sh.at[dst_v.at[j + b]], semS[b],
                             add=True)

    for b in range(NB):
        pltpu.make_async_copy(g[b], t_sh.at[dst_v.at[CPT + b - NB]],
                              semS[b]).wait()

    plsc.subcore_barrier()
    pltpu.sync_copy(t_sh.at[rng], sbuf)
    pltpu.sync_copy(sbuf, pout_hbm.at[c].at[rng])


@functools.partial(
    pl.kernel,
    out_type=jax.ShapeDtypeStruct((NCORES, NPAD, D_OUT), jnp.float32),
    mesh=_mesh,
    scratch_types=[
        pltpu.VMEM((CPT, CHUNK), jnp.int32),
        pltpu.VMEM((CHUNK, D_OUT), jnp.float32),
        pltpu.VMEM((RPS, D_OUT), jnp.float32),
        pltpu.VMEM_SHARED((NPAD, D_OUT), jnp.float32),
        pltpu.SemaphoreType.DMA,
    ],
    compiler_params=_sc_params,
)
def _deg_scatter(dst_hbm, out_hbm, dst_v, obuf, zbuf, t_sh, sem):
    c = lax.axis_index("c")
    sub = lax.axis_index("s")
    w = c * NSUB + sub
    pltpu.sync_copy(dst_hbm.at[w], dst_v)

    @pl.loop(0, CHUNK)
    def _(i):
        obuf[i, :] = jnp.ones((16,), jnp.float32)

    _zero_rows(zbuf)
    pltpu.sync_copy(zbuf, t_sh.at[pl.ds(pl.multiple_of(sub * RPS, 8), RPS)])
    plsc.subcore_barrier()

    @pl.loop(0, CPT // 8)
    def _(i):
        j = i * 8
        for b in range(8):
            pltpu.async_copy(obuf, t_sh.at[dst_v.at[j + b]], sem, add=True)
        for b in range(8):
            pltpu.make_async_copy(obuf, t_sh.at[dst_v.at[j + b]],
                                  sem).wait()

    plsc.subcore_barrier()
    pltpu.sync_copy(t_sh.at[pl.ds(pl.multiple_of(sub * RPS, 8), RPS)], zbuf)
    pltpu.sync_copy(zbuf, out_hbm.at[c].at[pl.ds(pl.multiple_of(sub * RPS, 8), RPS)])


def _mlp_prep_body(x_ref, w1_ref, b1_ref, w2_ref, b2_ref, pd_ref,
                   s0_ref, cs_ref, bs_ref, ch_ref, bh_ref):
    h = jnp.dot(x_ref[...], w1_ref[...], preferred_element_type=jnp.float32)
    h = jnp.maximum(h + b1_ref[...], 0.0)
    h0 = jnp.dot(h, w2_ref[...], preferred_element_type=jnp.float32) + b2_ref[...]
    pd = pd_ref[...]
    d = pd[0, :, 0:1] + pd[1, :, 0:1] + 1.0
    dinv = lax.rsqrt(d)
    rows = lax.broadcasted_iota(jnp.int32, (NPAD, 1), 0)
    dinv = jnp.where(rows < N, dinv, 0.0)
    s0_ref[...] = dinv * h0
    cs_ref[...] = jnp.broadcast_to((1.0 - ALPHA) * dinv * dinv, (NPAD, D_OUT))
    bs_ref[...] = (ALPHA * dinv) * h0
    ch_ref[...] = jnp.broadcast_to((1.0 - ALPHA) * dinv, (NPAD, D_OUT))
    bh_ref[...] = ALPHA * h0


def _mlp_prep(xp, W1, b1, W2, b2, pdeg):
    sh = jax.ShapeDtypeStruct((NPAD, D_OUT), jnp.float32)
    return pl.pallas_call(
        _mlp_prep_body,
        out_shape=(sh, sh, sh, sh, sh),
    )(xp, W1, b1.reshape(1, D_HID), W2, b2.reshape(1, D_OUT), pdeg)


def _update_body(p_ref, s_ref, c_ref, b_ref, o_ref):
    p = p_ref[...]
    o_ref[...] = c_ref[...] * (p[0] + p[1] + s_ref[...]) + b_ref[...]


def _update(p, s, coeff, bias):
    return pl.pallas_call(
        _update_body,
        out_shape=jax.ShapeDtypeStruct((NPAD, D_OUT), jnp.float32),
    )(p, s, coeff, bias)


def kernel(x, edge_index, W1, b1, W2, b2):
    xp = jnp.pad(x, ((0, NPAD - N), (0, 0)))
    pad = jnp.full((EPAD - E,), TRASH, jnp.int32)
    src3 = jnp.concatenate([edge_index[0], pad]).reshape(NTILES, CPT, CHUNK)
    dst3 = jnp.concatenate([edge_index[1], pad]).reshape(NTILES, CPT, CHUNK)

    pdeg = _deg_scatter(dst3)
    s0, cs, bs, ch, bh = _mlp_prep(xp, W1, b1, W2, b2, pdeg)

    zp = jnp.zeros((NCORES, NPAD, D_OUT), jnp.float32)
    zs = jnp.zeros((NPAD, D_OUT), jnp.float32)
    p, s = _hop(src3, dst3, zp, zs, cs, s0)
    for _ in range(K - 1):
        p, s = _hop(src3, dst3, p, s, cs, bs)
    return _update(p, s, ch, bh)[:N]
